# baseline (speedup 1.0000x reference)
import functools

import jax
import jax.numpy as jnp
from jax import lax
from jax.experimental import pallas as pl
from jax.experimental.pallas import tpu as pltpu
from jax.experimental.pallas import tpu_sc as plsc

NEG_SLOPE = 0.2
L = 16
NC = 2
NS = 16
NW = NC * NS
GROUP = 128


def _leaky(x):
    return jnp.where(x >= 0, x, NEG_SLOPE * x)


def _dense_body(x_ref, w_ref, al_ref, ar_ref, h_ref, el_ref, er_ref, gm_ref):
    h = jnp.dot(x_ref[...], w_ref[...], preferred_element_type=jnp.float32)
    h_ref[...] = h
    el = jnp.sum(h * al_ref[...], axis=-1, keepdims=True)
    el_ref[...] = el
    er_ref[...] = jnp.sum(h * ar_ref[...], axis=-1, keepdims=True)
    gm_ref[...] = jnp.full((1, L), jnp.max(el), jnp.float32)


def _dense(x, W, al, ar):
    N, _ = x.shape
    H = W.shape[1]
    return pl.pallas_call(
        _dense_body,
        out_shape=(
            jax.ShapeDtypeStruct((N, H), jnp.float32),
            jax.ShapeDtypeStruct((N, 1), jnp.float32),
            jax.ShapeDtypeStruct((N, 1), jnp.float32),
            jax.ShapeDtypeStruct((1, L), jnp.float32),
        ),
    )(x, W, al.reshape(1, H), ar.reshape(1, H))


def _combine_dense_body(pa_ref, pb_ref, b_ref, w_ref, al_ref, ar_ref,
                        h_ref, el_ref, er_ref, gm_ref):
    x = pa_ref[...] + pb_ref[...] + b_ref[...]
    h = jnp.dot(x, w_ref[...], preferred_element_type=jnp.float32)
    h_ref[...] = h
    el = jnp.sum(h * al_ref[...], axis=-1, keepdims=True)
    el_ref[...] = el
    er_ref[...] = jnp.sum(h * ar_ref[...], axis=-1, keepdims=True)
    gm_ref[...] = jnp.full((1, L), jnp.max(el), jnp.float32)


def _combine_dense(pa, pb, b, W, al, ar):
    N, _ = pa.shape
    H = W.shape[1]
    return pl.pallas_call(
        _combine_dense_body,
        out_shape=(
            jax.ShapeDtypeStruct((N, H), jnp.float32),
            jax.ShapeDtypeStruct((N, 1), jnp.float32),
            jax.ShapeDtypeStruct((N, 1), jnp.float32),
            jax.ShapeDtypeStruct((1, L), jnp.float32),
        ),
    )(pa, pb, b.reshape(1, H), W, al.reshape(1, H), ar.reshape(1, H))


def _dsum_body(d_ref, o_ref):
    o_ref[...] = d_ref[0:1, :] + d_ref[1:2, :]


def _dsum(dpart):
    n_pad = dpart.shape[1]
    return pl.pallas_call(
        _dsum_body,
        out_shape=jax.ShapeDtypeStruct((1, n_pad), jnp.float32),
    )(dpart)


def _combine_body(pa_ref, pb_ref, b_ref, o_ref):
    o_ref[...] = pa_ref[...] + pb_ref[...] + b_ref[...]


def _combine(pa, pb, b):
    N, H = pa.shape
    return pl.pallas_call(
        _combine_body,
        out_shape=jax.ShapeDtypeStruct((N, H), jnp.float32),
    )(pa, pb, b.reshape(1, H))


def _make_pass1(n_pad, gtot, gp_w):
    mesh = plsc.VectorSubcoreMesh(core_axis_name="c", subcore_axis_name="s")
    nw16 = n_pad // NS

    def body(src_ref, dst_ref, el_ref, er_ref, gm_ref,
             ex_ref, dpart_ref,
             src_v, dst_v, ex_v, zd_v, gm_v,
             elg_v, erg_v,
             el_sh, er_sh, denom_sh):
        cid = lax.axis_index("c")
        sid = lax.axis_index("s")
        wid = cid * NS + sid

        for i in range(0, nw16, L):
            zd_v[pl.ds(i, L)] = jnp.zeros((L,), jnp.float32)
        pltpu.sync_copy(zd_v, denom_sh.at[pl.ds(sid * nw16, nw16)])

        @pl.when(sid == 0)
        def _stage():
            pltpu.sync_copy(el_ref, el_sh)
            pltpu.sync_copy(er_ref, er_sh)
        pltpu.sync_copy(src_ref.at[pl.ds(wid * gp_w, gp_w)], src_v)
        pltpu.sync_copy(dst_ref.at[pl.ds(wid * gp_w, gp_w)], dst_v)
        pltpu.sync_copy(gm_ref, gm_v)
        plsc.subcore_barrier()
        gmax = gm_v[0, pl.ds(0, L)]

        def g_body(g, carry):
            pltpu.sync_copy(el_sh.at[src_v.at[g]], elg_v)
            pltpu.sync_copy(er_sh.at[dst_v.at[g]], erg_v)
            for i in range(GROUP // L):
                elv = elg_v[pl.ds(i * L, L)]
                erv = erg_v[pl.ds(i * L, L)]
                e = _leaky(elv + erv)
                mhat = _leaky(gmax + erv)
                ex_v[g, pl.ds(i * L, L)] = jnp.exp(e - mhat)
            pltpu.sync_copy(ex_v.at[g], denom_sh.at[dst_v.at[g]], add=True)
            return carry
        lax.fori_loop(0, gp_w, g_body, 0)

        pltpu.sync_copy(ex_v, ex_ref.at[pl.ds(wid * gp_w, gp_w)])
        plsc.subcore_barrier()
        pltpu.sync_copy(denom_sh.at[pl.ds(sid * nw16, nw16)],
                        dpart_ref.at[cid, pl.ds(sid * nw16, nw16)])

    return pl.kernel(
        body,
        out_type=(
            jax.ShapeDtypeStruct((gtot, GROUP), jnp.float32),
            jax.ShapeDtypeStruct((NC, n_pad), jnp.float32),
        ),
        mesh=mesh,
        scratch_types=[
            pltpu.VMEM((gp_w, GROUP), jnp.int32),
            pltpu.VMEM((gp_w, GROUP), jnp.int32),
            pltpu.VMEM((gp_w, GROUP), jnp.float32),
            pltpu.VMEM((nw16,), jnp.float32),
            pltpu.VMEM((1, L), jnp.float32),
            pltpu.VMEM((GROUP,), jnp.float32),
            pltpu.VMEM((GROUP,), jnp.float32),
            pltpu.VMEM_SHARED((n_pad,), jnp.float32),
            pltpu.VMEM_SHARED((n_pad,), jnp.float32),
            pltpu.VMEM_SHARED((n_pad,), jnp.float32),
        ],
    )


CG = 16
GP0 = 48


def _make_pass2(n_pad, gtot, gp_w, H):
    mesh = plsc.VectorSubcoreMesh(core_axis_name="c", subcore_axis_name="s")
    rows_pt = n_pad // NS

    def body(src_ref, dst_ref, exh_ref, dn_ref, h_ref,
             opart_ref,
             src_c, dst_c, ex_c, rows_v, dnl_v, rec_v,
             out_sh, gsem0, gsem1, ssem0, ssem1):
        cid = lax.axis_index("c")
        sid = lax.axis_index("s")
        wid = cid * NS + sid

        def z_body(r, carry):
            for i in range(H // L):
                rows_v[0, r, pl.ds(i * L, L)] = jnp.zeros((L,), jnp.float32)
            return carry
        lax.fori_loop(0, GROUP, z_body, 0)
        for j in range(0, rows_pt, GROUP):
            pltpu.sync_copy(rows_v.at[0],
                            out_sh.at[pl.ds(sid * rows_pt + j, GROUP)])
        plsc.subcore_barrier()

        def _rowscale(b, a_of_i):
            def e_body(i, c2):
                a16 = a_of_i(i)
                for l in range(L):
                    e = i * L + l
                    a = a16[l]
                    for k in range(H // L):
                        rows_v[b, e, pl.ds(k * L, L)] = (
                            rows_v[b, e, pl.ds(k * L, L)] * a)
                return c2
            lax.fori_loop(0, GROUP // L, e_body, 0)

        def _wait(b, sem):
            pltpu.make_async_copy(h_ref.at[pl.ds(0, GROUP)],
                                  rows_v.at[b], sem).wait()

        gp_me = jnp.where(cid == 0, GP0, 2 * gp_w - GP0)
        base_g = sid * (2 * gp_w) + jnp.where(cid == 0, 0, GP0)

        def chunk_body(c, carry):
            base = base_g + c * CG
            pltpu.sync_copy(src_ref.at[pl.ds(base, CG)], src_c)
            pltpu.sync_copy(dst_ref.at[pl.ds(base, CG)], dst_c)
            pltpu.sync_copy(exh_ref.at[pl.ds(base, CG)], ex_c)
            pltpu.async_copy(h_ref.at[src_c.at[0]], rows_v.at[0], gsem0)

            def pair_body(j, c2):
                g0 = 2 * j
                g1 = 2 * j + 1
                _wait(0, gsem0)

                @pl.when(c + j > 0)
                def _ws1():
                    _wait(1, ssem1)
                pltpu.async_copy(h_ref.at[src_c.at[g1]], rows_v.at[1], gsem1)
                _rowscale(0, lambda i: ex_c[g0, pl.ds(i * L, L)])
                pltpu.async_copy(rows_v.at[0], out_sh.at[dst_c.at[g0]],
                                 ssem0, add=True)
                _wait(1, gsem1)
                _wait(0, ssem0)

                @pl.when(j + 1 < CG // 2)
                def _prefetch():
                    pltpu.async_copy(h_ref.at[src_c.at[g0 + 2]],
                                     rows_v.at[0], gsem0)
                _rowscale(1, lambda i: ex_c[g1, pl.ds(i * L, L)])
                pltpu.async_copy(rows_v.at[1], out_sh.at[dst_c.at[g1]],
                                 ssem1, add=True)
                return c2
            lax.fori_loop(0, CG // 2, pair_body, 0)
            return carry
        lax.fori_loop(0, gp_me // CG, chunk_body, 0)
        _wait(1, ssem1)
        plsc.subcore_barrier()

        pltpu.sync_copy(dn_ref.at[pl.ds(sid * rows_pt, rows_pt)], dnl_v)
        for j in range(0, rows_pt, GROUP):
            pltpu.sync_copy(out_sh.at[pl.ds(sid * rows_pt + j, GROUP)],
                            rows_v.at[0])
            for i in range(GROUP // L):
                rec_v[pl.ds(i * L, L)] = 1.0 / jnp.maximum(
                    dnl_v[pl.ds(j + i * L, L)], 1e-37)
            _rowscale(0, lambda i: rec_v[pl.ds(i * L, L)])
            pltpu.sync_copy(rows_v.at[0],
                            opart_ref.at[cid, pl.ds(sid * rows_pt + j, GROUP)])

    return pl.kernel(
        body,
        out_type=jax.ShapeDtypeStruct((NC, n_pad, H), jnp.float32),
        mesh=mesh,
        scratch_types=[
            pltpu.VMEM((CG, GROUP), jnp.int32),
            pltpu.VMEM((CG, GROUP), jnp.int32),
            pltpu.VMEM((CG, GROUP), jnp.float32),
            pltpu.VMEM((2, GROUP, H), jnp.float32),
            pltpu.VMEM((rows_pt,), jnp.float32),
            pltpu.VMEM((GROUP,), jnp.float32),
            pltpu.VMEM_SHARED((n_pad, H), jnp.float32),
            pltpu.SemaphoreType.DMA,
            pltpu.SemaphoreType.DMA,
            pltpu.SemaphoreType.DMA,
            pltpu.SemaphoreType.DMA,
        ],
    )


def kernel(feat, edge_index, efeat, W1, al1, ar1, b1, W2, al2, ar2, b2,
           W3, al3, ar3, b3):
    N, D = feat.shape
    H = W1.shape[1]
    E = edge_index.shape[1]

    chunk = NW * GROUP * 8
    e_pad = ((E + chunk - 1) // chunk) * chunk
    gtot = e_pad // GROUP
    gp_w = gtot // NW
    n_pad = ((N + 1 + 255) // 256) * 256

    src = edge_index[0].astype(jnp.int32)
    dst = edge_index[1].astype(jnp.int32)
    pad = e_pad - E
    src2d = jnp.concatenate([src, jnp.zeros((pad,), jnp.int32)]).reshape(gtot, GROUP)
    dst2d = jnp.concatenate([dst, jnp.full((pad,), N, jnp.int32)]).reshape(gtot, GROUP)

    p1 = _make_pass1(n_pad, gtot, gp_w)
    p2 = _make_pass2(n_pad, gtot, gp_w, H)

    def layer(h, el, er, gm):
        el_p = jnp.pad(el[:, 0], (0, n_pad - N))
        er_p = jnp.pad(er[:, 0], (0, n_pad - N))
        ex, dpart = p1(src2d, dst2d, el_p, er_p, gm)
        dn = _dsum(dpart).reshape(n_pad)
        op = p2(src2d, dst2d, ex, dn, h)
        return op[:, :N]

    h, el, er, gm = _dense(feat, W1, al1, ar1)
    op = layer(h, el, er, gm)
    h, el, er, gm = _combine_dense(op[0], op[1], b1, W2, al2, ar2)
    op = layer(h, el, er, gm)
    h, el, er, gm = _combine_dense(op[0], op[1], b2, W3, al3, ar3)
    op = layer(h, el, er, gm)
    out = _combine(op[0], op[1], b3)
    return out[:, None, :]


# speedup vs baseline: 1.2258x; 1.2258x over previous
import functools

import jax
import jax.numpy as jnp
from jax import lax
from jax.experimental import pallas as pl
from jax.experimental.pallas import tpu as pltpu
from jax.experimental.pallas import tpu_sc as plsc

NEG_SLOPE = 0.2
L = 16
NC = 2
NS = 16
NW = NC * NS
GROUP = 128


def _leaky(x):
    return jnp.where(x >= 0, x, NEG_SLOPE * x)


def _dense_body(x_ref, w_ref, al_ref, ar_ref, h_ref, el_ref, er_ref, gm_ref):
    h = jnp.dot(x_ref[...], w_ref[...], preferred_element_type=jnp.float32)
    h_ref[...] = h
    el = jnp.sum(h * al_ref[...], axis=-1, keepdims=True)
    el_ref[...] = el
    er_ref[...] = jnp.sum(h * ar_ref[...], axis=-1, keepdims=True)
    gm_ref[...] = jnp.full((1, L), jnp.max(el), jnp.float32)


def _dense(x, W, al, ar):
    N, _ = x.shape
    H = W.shape[1]
    return pl.pallas_call(
        _dense_body,
        out_shape=(
            jax.ShapeDtypeStruct((N, H), jnp.float32),
            jax.ShapeDtypeStruct((N, 1), jnp.float32),
            jax.ShapeDtypeStruct((N, 1), jnp.float32),
            jax.ShapeDtypeStruct((1, L), jnp.float32),
        ),
    )(x, W, al.reshape(1, H), ar.reshape(1, H))


def _combine_dense_body(pa_ref, pb_ref, b_ref, w_ref, al_ref, ar_ref,
                        h_ref, el_ref, er_ref, gm_ref):
    x = pa_ref[...] + pb_ref[...] + b_ref[...]
    h = jnp.dot(x, w_ref[...], preferred_element_type=jnp.float32)
    h_ref[...] = h
    el = jnp.sum(h * al_ref[...], axis=-1, keepdims=True)
    el_ref[...] = el
    er_ref[...] = jnp.sum(h * ar_ref[...], axis=-1, keepdims=True)
    gm_ref[...] = jnp.full((1, L), jnp.max(el), jnp.float32)


def _combine_dense(pa, pb, b, W, al, ar):
    N, _ = pa.shape
    H = W.shape[1]
    return pl.pallas_call(
        _combine_dense_body,
        out_shape=(
            jax.ShapeDtypeStruct((N, H), jnp.float32),
            jax.ShapeDtypeStruct((N, 1), jnp.float32),
            jax.ShapeDtypeStruct((N, 1), jnp.float32),
            jax.ShapeDtypeStruct((1, L), jnp.float32),
        ),
    )(pa, pb, b.reshape(1, H), W, al.reshape(1, H), ar.reshape(1, H))


def _dsum_body(d_ref, o_ref):
    o_ref[...] = d_ref[0:1, :] + d_ref[1:2, :]


def _dsum(dpart):
    n_pad = dpart.shape[1]
    return pl.pallas_call(
        _dsum_body,
        out_shape=jax.ShapeDtypeStruct((1, n_pad), jnp.float32),
    )(dpart)


def _combine_body(pa_ref, pb_ref, b_ref, o_ref):
    o_ref[...] = pa_ref[...] + pb_ref[...] + b_ref[...]


def _combine(pa, pb, b):
    N, H = pa.shape
    return pl.pallas_call(
        _combine_body,
        out_shape=jax.ShapeDtypeStruct((N, H), jnp.float32),
    )(pa, pb, b.reshape(1, H))


def _make_pass1(n_pad, gtot, gp_w):
    mesh = plsc.VectorSubcoreMesh(core_axis_name="c", subcore_axis_name="s")
    nw16 = n_pad // NS

    def body(src_ref, dst_ref, el_ref, er_ref, gm_ref,
             ex_ref, dpart_ref,
             src_v, dst_v, ex_v, zd_v, gm_v,
             elg_v, erg_v,
             el_sh, er_sh, denom_sh):
        cid = lax.axis_index("c")
        sid = lax.axis_index("s")
        wid = cid * NS + sid

        for i in range(0, nw16, L):
            zd_v[pl.ds(i, L)] = jnp.zeros((L,), jnp.float32)
        pltpu.sync_copy(zd_v, denom_sh.at[pl.ds(sid * nw16, nw16)])

        @pl.when(sid == 0)
        def _stage():
            pltpu.sync_copy(el_ref, el_sh)
            pltpu.sync_copy(er_ref, er_sh)
        pltpu.sync_copy(src_ref.at[pl.ds(wid * gp_w, gp_w)], src_v)
        pltpu.sync_copy(dst_ref.at[pl.ds(wid * gp_w, gp_w)], dst_v)
        pltpu.sync_copy(gm_ref, gm_v)
        plsc.subcore_barrier()
        gmax = gm_v[0, pl.ds(0, L)]

        def g_body(g, carry):
            pltpu.sync_copy(el_sh.at[src_v.at[g]], elg_v)
            pltpu.sync_copy(er_sh.at[dst_v.at[g]], erg_v)
            for i in range(GROUP // L):
                elv = elg_v[pl.ds(i * L, L)]
                erv = erg_v[pl.ds(i * L, L)]
                e = _leaky(elv + erv)
                mhat = _leaky(gmax + erv)
                ex_v[g, pl.ds(i * L, L)] = jnp.exp(e - mhat)
            pltpu.sync_copy(ex_v.at[g], denom_sh.at[dst_v.at[g]], add=True)
            return carry
        lax.fori_loop(0, gp_w, g_body, 0)

        pltpu.sync_copy(ex_v, ex_ref.at[pl.ds(wid * gp_w, gp_w)])
        plsc.subcore_barrier()
        pltpu.sync_copy(denom_sh.at[pl.ds(sid * nw16, nw16)],
                        dpart_ref.at[cid, pl.ds(sid * nw16, nw16)])

    return pl.kernel(
        body,
        out_type=(
            jax.ShapeDtypeStruct((gtot, GROUP), jnp.float32),
            jax.ShapeDtypeStruct((NC, n_pad), jnp.float32),
        ),
        mesh=mesh,
        scratch_types=[
            pltpu.VMEM((gp_w, GROUP), jnp.int32),
            pltpu.VMEM((gp_w, GROUP), jnp.int32),
            pltpu.VMEM((gp_w, GROUP), jnp.float32),
            pltpu.VMEM((nw16,), jnp.float32),
            pltpu.VMEM((1, L), jnp.float32),
            pltpu.VMEM((GROUP,), jnp.float32),
            pltpu.VMEM((GROUP,), jnp.float32),
            pltpu.VMEM_SHARED((n_pad,), jnp.float32),
            pltpu.VMEM_SHARED((n_pad,), jnp.float32),
            pltpu.VMEM_SHARED((n_pad,), jnp.float32),
        ],
    )


CG = 16
GP0 = 112


def _make_pass2(n_pad, gtot, gp_w, H):
    mesh = plsc.VectorSubcoreMesh(core_axis_name="c", subcore_axis_name="s")
    rows_pt = n_pad // NS

    def body(src_ref, dst_ref, exh_ref, dn_ref, h_ref,
             opart_ref,
             src_c, dst_c, ex_c, rows_v, dnl_v, rec_v,
             out_sh, gsem0, gsem1, ssem0, ssem1):
        cid = lax.axis_index("c")
        sid = lax.axis_index("s")
        wid = cid * NS + sid

        def z_body(r, carry):
            for i in range(H // L):
                rows_v[0, r, pl.ds(i * L, L)] = jnp.zeros((L,), jnp.float32)
            return carry
        lax.fori_loop(0, GROUP, z_body, 0)
        for j in range(0, rows_pt, GROUP):
            pltpu.sync_copy(rows_v.at[0],
                            out_sh.at[pl.ds(sid * rows_pt + j, GROUP)])
        plsc.subcore_barrier()

        def _rowscale(b, a_of_i):
            def e_body(i, c2):
                a16 = a_of_i(i)
                for l in range(L):
                    e = i * L + l
                    a = a16[l]
                    for k in range(H // L):
                        rows_v[b, e, pl.ds(k * L, L)] = (
                            rows_v[b, e, pl.ds(k * L, L)] * a)
                return c2
            lax.fori_loop(0, GROUP // L, e_body, 0)

        def _wait(b, sem):
            pltpu.make_async_copy(h_ref.at[pl.ds(0, GROUP)],
                                  rows_v.at[b], sem).wait()

        gp_me = jnp.where(cid == 0, GP0, 2 * gp_w - GP0)
        base_g = sid * (2 * gp_w) + jnp.where(cid == 0, 0, GP0)

        def chunk_body(c, carry):
            base = base_g + c * CG
            pltpu.sync_copy(src_ref.at[pl.ds(base, CG)], src_c)
            pltpu.sync_copy(dst_ref.at[pl.ds(base, CG)], dst_c)
            pltpu.sync_copy(exh_ref.at[pl.ds(base, CG)], ex_c)
            pltpu.async_copy(h_ref.at[src_c.at[0]], rows_v.at[0], gsem0)

            def pair_body(j, c2):
                g0 = 2 * j
                g1 = 2 * j + 1
                _wait(0, gsem0)

                @pl.when(c + j > 0)
                def _ws1():
                    _wait(1, ssem1)
                pltpu.async_copy(h_ref.at[src_c.at[g1]], rows_v.at[1], gsem1)
                _rowscale(0, lambda i: ex_c[g0, pl.ds(i * L, L)])
                pltpu.async_copy(rows_v.at[0], out_sh.at[dst_c.at[g0]],
                                 ssem0, add=True)
                _wait(1, gsem1)
                _wait(0, ssem0)

                @pl.when(j + 1 < CG // 2)
                def _prefetch():
                    pltpu.async_copy(h_ref.at[src_c.at[g0 + 2]],
                                     rows_v.at[0], gsem0)
                _rowscale(1, lambda i: ex_c[g1, pl.ds(i * L, L)])
                pltpu.async_copy(rows_v.at[1], out_sh.at[dst_c.at[g1]],
                                 ssem1, add=True)
                return c2
            lax.fori_loop(0, CG // 2, pair_body, 0)
            return carry
        lax.fori_loop(0, gp_me // CG, chunk_body, 0)
        _wait(1, ssem1)
        plsc.subcore_barrier()

        pltpu.sync_copy(dn_ref.at[pl.ds(sid * rows_pt, rows_pt)], dnl_v)
        for j in range(0, rows_pt, GROUP):
            pltpu.sync_copy(out_sh.at[pl.ds(sid * rows_pt + j, GROUP)],
                            rows_v.at[0])
            for i in range(GROUP // L):
                rec_v[pl.ds(i * L, L)] = 1.0 / jnp.maximum(
                    dnl_v[pl.ds(j + i * L, L)], 1e-37)
            _rowscale(0, lambda i: rec_v[pl.ds(i * L, L)])
            pltpu.sync_copy(rows_v.at[0],
                            opart_ref.at[cid, pl.ds(sid * rows_pt + j, GROUP)])

    return pl.kernel(
        body,
        out_type=jax.ShapeDtypeStruct((NC, n_pad, H), jnp.float32),
        mesh=mesh,
        scratch_types=[
            pltpu.VMEM((CG, GROUP), jnp.int32),
            pltpu.VMEM((CG, GROUP), jnp.int32),
            pltpu.VMEM((CG, GROUP), jnp.float32),
            pltpu.VMEM((2, GROUP, H), jnp.float32),
            pltpu.VMEM((rows_pt,), jnp.float32),
            pltpu.VMEM((GROUP,), jnp.float32),
            pltpu.VMEM_SHARED((n_pad, H), jnp.float32),
            pltpu.SemaphoreType.DMA,
            pltpu.SemaphoreType.DMA,
            pltpu.SemaphoreType.DMA,
            pltpu.SemaphoreType.DMA,
        ],
    )


def kernel(feat, edge_index, efeat, W1, al1, ar1, b1, W2, al2, ar2, b2,
           W3, al3, ar3, b3):
    N, D = feat.shape
    H = W1.shape[1]
    E = edge_index.shape[1]

    chunk = NW * GROUP * 8
    e_pad = ((E + chunk - 1) // chunk) * chunk
    gtot = e_pad // GROUP
    gp_w = gtot // NW
    n_pad = ((N + 1 + 255) // 256) * 256

    src = edge_index[0].astype(jnp.int32)
    dst = edge_index[1].astype(jnp.int32)
    pad = e_pad - E
    src2d = jnp.concatenate([src, jnp.zeros((pad,), jnp.int32)]).reshape(gtot, GROUP)
    dst2d = jnp.concatenate([dst, jnp.full((pad,), N, jnp.int32)]).reshape(gtot, GROUP)

    p1 = _make_pass1(n_pad, gtot, gp_w)
    p2 = _make_pass2(n_pad, gtot, gp_w, H)

    def layer(h, el, er, gm):
        el_p = jnp.pad(el[:, 0], (0, n_pad - N))
        er_p = jnp.pad(er[:, 0], (0, n_pad - N))
        ex, dpart = p1(src2d, dst2d, el_p, er_p, gm)
        dn = _dsum(dpart).reshape(n_pad)
        op = p2(src2d, dst2d, ex, dn, h)
        return op[:, :N]

    h, el, er, gm = _dense(feat, W1, al1, ar1)
    op = layer(h, el, er, gm)
    h, el, er, gm = _combine_dense(op[0], op[1], b1, W2, al2, ar2)
    op = layer(h, el, er, gm)
    h, el, er, gm = _combine_dense(op[0], op[1], b2, W3, al3, ar3)
    op = layer(h, el, er, gm)
    out = _combine(op[0], op[1], b3)
    return out[:, None, :]


# speedup vs baseline: 1.2650x; 1.0319x over previous
import functools

import jax
import jax.numpy as jnp
from jax import lax
from jax.experimental import pallas as pl
from jax.experimental.pallas import tpu as pltpu
from jax.experimental.pallas import tpu_sc as plsc

NEG_SLOPE = 0.2
L = 16
NC = 2
NS = 16
NW = NC * NS
GROUP = 128


def _leaky(x):
    return jnp.where(x >= 0, x, NEG_SLOPE * x)


def _dense_body(x_ref, w_ref, al_ref, ar_ref, h_ref, el_ref, er_ref, gm_ref):
    h = jnp.dot(x_ref[...], w_ref[...], preferred_element_type=jnp.float32)
    h_ref[...] = h
    el = jnp.sum(h * al_ref[...], axis=-1, keepdims=True)
    el_ref[...] = el
    er_ref[...] = jnp.sum(h * ar_ref[...], axis=-1, keepdims=True)
    gm_ref[...] = jnp.full((1, L), jnp.max(el), jnp.float32)


def _dense(x, W, al, ar):
    N, _ = x.shape
    H = W.shape[1]
    return pl.pallas_call(
        _dense_body,
        out_shape=(
            jax.ShapeDtypeStruct((N, H), jnp.float32),
            jax.ShapeDtypeStruct((N, 1), jnp.float32),
            jax.ShapeDtypeStruct((N, 1), jnp.float32),
            jax.ShapeDtypeStruct((1, L), jnp.float32),
        ),
    )(x, W, al.reshape(1, H), ar.reshape(1, H))


def _combine_dense_body(pa_ref, pb_ref, b_ref, w_ref, al_ref, ar_ref,
                        h_ref, el_ref, er_ref, gm_ref):
    x = pa_ref[...] + pb_ref[...] + b_ref[...]
    h = jnp.dot(x, w_ref[...], preferred_element_type=jnp.float32)
    h_ref[...] = h
    el = jnp.sum(h * al_ref[...], axis=-1, keepdims=True)
    el_ref[...] = el
    er_ref[...] = jnp.sum(h * ar_ref[...], axis=-1, keepdims=True)
    gm_ref[...] = jnp.full((1, L), jnp.max(el), jnp.float32)


def _combine_dense(pa, pb, b, W, al, ar):
    N, _ = pa.shape
    H = W.shape[1]
    return pl.pallas_call(
        _combine_dense_body,
        out_shape=(
            jax.ShapeDtypeStruct((N, H), jnp.float32),
            jax.ShapeDtypeStruct((N, 1), jnp.float32),
            jax.ShapeDtypeStruct((N, 1), jnp.float32),
            jax.ShapeDtypeStruct((1, L), jnp.float32),
        ),
    )(pa, pb, b.reshape(1, H), W, al.reshape(1, H), ar.reshape(1, H))


def _dsum_body(d_ref, o_ref):
    o_ref[...] = d_ref[0:1, :] + d_ref[1:2, :]


def _dsum(dpart):
    n_pad = dpart.shape[1]
    return pl.pallas_call(
        _dsum_body,
        out_shape=jax.ShapeDtypeStruct((1, n_pad), jnp.float32),
    )(dpart)


def _combine_body(pa_ref, pb_ref, b_ref, o_ref):
    o_ref[...] = pa_ref[...] + pb_ref[...] + b_ref[...]


def _combine(pa, pb, b):
    N, H = pa.shape
    return pl.pallas_call(
        _combine_body,
        out_shape=jax.ShapeDtypeStruct((N, H), jnp.float32),
    )(pa, pb, b.reshape(1, H))


def _make_pass1(n_pad, gtot, gp_w):
    mesh = plsc.VectorSubcoreMesh(core_axis_name="c", subcore_axis_name="s")
    nw16 = n_pad // NS

    def body(src_ref, dst_ref, el_ref, er_ref, gm_ref,
             ex_ref, dpart_ref,
             src_v, dst_v, ex_v, zd_v, gm_v,
             elg_v, erg_v,
             el_sh, er_sh, denom_sh):
        cid = lax.axis_index("c")
        sid = lax.axis_index("s")
        wid = cid * NS + sid

        for i in range(0, nw16, L):
            zd_v[pl.ds(i, L)] = jnp.zeros((L,), jnp.float32)
        pltpu.sync_copy(zd_v, denom_sh.at[pl.ds(sid * nw16, nw16)])

        @pl.when(sid == 0)
        def _stage():
            pltpu.sync_copy(el_ref, el_sh)
            pltpu.sync_copy(er_ref, er_sh)
        pltpu.sync_copy(src_ref.at[pl.ds(wid * gp_w, gp_w)], src_v)
        pltpu.sync_copy(dst_ref.at[pl.ds(wid * gp_w, gp_w)], dst_v)
        pltpu.sync_copy(gm_ref, gm_v)
        plsc.subcore_barrier()
        gmax = gm_v[0, pl.ds(0, L)]

        def g_body(g, carry):
            pltpu.sync_copy(el_sh.at[src_v.at[g]], elg_v)
            pltpu.sync_copy(er_sh.at[dst_v.at[g]], erg_v)
            for i in range(GROUP // L):
                elv = elg_v[pl.ds(i * L, L)]
                erv = erg_v[pl.ds(i * L, L)]
                e = _leaky(elv + erv)
                mhat = _leaky(gmax + erv)
                ex_v[g, pl.ds(i * L, L)] = jnp.exp(e - mhat)
            pltpu.sync_copy(ex_v.at[g], denom_sh.at[dst_v.at[g]], add=True)
            return carry
        lax.fori_loop(0, gp_w, g_body, 0)

        pltpu.sync_copy(ex_v, ex_ref.at[pl.ds(wid * gp_w, gp_w)])
        plsc.subcore_barrier()
        pltpu.sync_copy(denom_sh.at[pl.ds(sid * nw16, nw16)],
                        dpart_ref.at[cid, pl.ds(sid * nw16, nw16)])

    return pl.kernel(
        body,
        out_type=(
            jax.ShapeDtypeStruct((gtot, GROUP), jnp.float32),
            jax.ShapeDtypeStruct((NC, n_pad), jnp.float32),
        ),
        mesh=mesh,
        scratch_types=[
            pltpu.VMEM((gp_w, GROUP), jnp.int32),
            pltpu.VMEM((gp_w, GROUP), jnp.int32),
            pltpu.VMEM((gp_w, GROUP), jnp.float32),
            pltpu.VMEM((nw16,), jnp.float32),
            pltpu.VMEM((1, L), jnp.float32),
            pltpu.VMEM((GROUP,), jnp.float32),
            pltpu.VMEM((GROUP,), jnp.float32),
            pltpu.VMEM_SHARED((n_pad,), jnp.float32),
            pltpu.VMEM_SHARED((n_pad,), jnp.float32),
            pltpu.VMEM_SHARED((n_pad,), jnp.float32),
        ],
    )


CG = 8
GP0 = 120


def _make_pass2(n_pad, gtot, gp_w, H):
    mesh = plsc.VectorSubcoreMesh(core_axis_name="c", subcore_axis_name="s")
    rows_pt = n_pad // NS

    def body(src_ref, dst_ref, exh_ref, dn_ref, h_ref,
             opart_ref,
             src_c, dst_c, ex_c, rows_v, dnl_v, rec_v,
             out_sh, gsem0, gsem1, ssem0, ssem1):
        cid = lax.axis_index("c")
        sid = lax.axis_index("s")
        wid = cid * NS + sid

        def z_body(r, carry):
            for i in range(H // L):
                rows_v[0, r, pl.ds(i * L, L)] = jnp.zeros((L,), jnp.float32)
            return carry
        lax.fori_loop(0, GROUP, z_body, 0)
        for j in range(0, rows_pt, GROUP):
            pltpu.sync_copy(rows_v.at[0],
                            out_sh.at[pl.ds(sid * rows_pt + j, GROUP)])
        plsc.subcore_barrier()

        def _rowscale(b, a_of_i):
            def e_body(i, c2):
                a16 = a_of_i(i)
                for l in range(L):
                    e = i * L + l
                    a = a16[l]
                    for k in range(H // L):
                        rows_v[b, e, pl.ds(k * L, L)] = (
                            rows_v[b, e, pl.ds(k * L, L)] * a)
                return c2
            lax.fori_loop(0, GROUP // L, e_body, 0)

        def _wait(b, sem):
            pltpu.make_async_copy(h_ref.at[pl.ds(0, GROUP)],
                                  rows_v.at[b], sem).wait()

        gp_me = jnp.where(cid == 0, GP0, 2 * gp_w - GP0)
        base_g = sid * (2 * gp_w) + jnp.where(cid == 0, 0, GP0)

        def chunk_body(c, carry):
            base = base_g + c * CG
            pltpu.sync_copy(src_ref.at[pl.ds(base, CG)], src_c)
            pltpu.sync_copy(dst_ref.at[pl.ds(base, CG)], dst_c)
            pltpu.sync_copy(exh_ref.at[pl.ds(base, CG)], ex_c)
            pltpu.async_copy(h_ref.at[src_c.at[0]], rows_v.at[0], gsem0)

            def pair_body(j, c2):
                g0 = 2 * j
                g1 = 2 * j + 1
                _wait(0, gsem0)

                @pl.when(c + j > 0)
                def _ws1():
                    _wait(1, ssem1)
                pltpu.async_copy(h_ref.at[src_c.at[g1]], rows_v.at[1], gsem1)
                _rowscale(0, lambda i: ex_c[g0, pl.ds(i * L, L)])
                pltpu.async_copy(rows_v.at[0], out_sh.at[dst_c.at[g0]],
                                 ssem0, add=True)
                _wait(1, gsem1)
                _wait(0, ssem0)

                @pl.when(j + 1 < CG // 2)
                def _prefetch():
                    pltpu.async_copy(h_ref.at[src_c.at[g0 + 2]],
                                     rows_v.at[0], gsem0)
                _rowscale(1, lambda i: ex_c[g1, pl.ds(i * L, L)])
                pltpu.async_copy(rows_v.at[1], out_sh.at[dst_c.at[g1]],
                                 ssem1, add=True)
                return c2
            lax.fori_loop(0, CG // 2, pair_body, 0)
            return carry
        lax.fori_loop(0, gp_me // CG, chunk_body, 0)
        _wait(1, ssem1)
        plsc.subcore_barrier()

        pltpu.sync_copy(dn_ref.at[pl.ds(sid * rows_pt, rows_pt)], dnl_v)
        for j in range(0, rows_pt, GROUP):
            pltpu.sync_copy(out_sh.at[pl.ds(sid * rows_pt + j, GROUP)],
                            rows_v.at[0])
            for i in range(GROUP // L):
                rec_v[pl.ds(i * L, L)] = 1.0 / jnp.maximum(
                    dnl_v[pl.ds(j + i * L, L)], 1e-37)
            _rowscale(0, lambda i: rec_v[pl.ds(i * L, L)])
            pltpu.sync_copy(rows_v.at[0],
                            opart_ref.at[cid, pl.ds(sid * rows_pt + j, GROUP)])

    return pl.kernel(
        body,
        out_type=jax.ShapeDtypeStruct((NC, n_pad, H), jnp.float32),
        mesh=mesh,
        scratch_types=[
            pltpu.VMEM((CG, GROUP), jnp.int32),
            pltpu.VMEM((CG, GROUP), jnp.int32),
            pltpu.VMEM((CG, GROUP), jnp.float32),
            pltpu.VMEM((2, GROUP, H), jnp.float32),
            pltpu.VMEM((rows_pt,), jnp.float32),
            pltpu.VMEM((GROUP,), jnp.float32),
            pltpu.VMEM_SHARED((n_pad, H), jnp.float32),
            pltpu.SemaphoreType.DMA,
            pltpu.SemaphoreType.DMA,
            pltpu.SemaphoreType.DMA,
            pltpu.SemaphoreType.DMA,
        ],
    )


def kernel(feat, edge_index, efeat, W1, al1, ar1, b1, W2, al2, ar2, b2,
           W3, al3, ar3, b3):
    N, D = feat.shape
    H = W1.shape[1]
    E = edge_index.shape[1]

    chunk = NW * GROUP * 8
    e_pad = ((E + chunk - 1) // chunk) * chunk
    gtot = e_pad // GROUP
    gp_w = gtot // NW
    n_pad = ((N + 1 + 255) // 256) * 256

    src = edge_index[0].astype(jnp.int32)
    dst = edge_index[1].astype(jnp.int32)
    pad = e_pad - E
    src2d = jnp.concatenate([src, jnp.zeros((pad,), jnp.int32)]).reshape(gtot, GROUP)
    dst2d = jnp.concatenate([dst, jnp.full((pad,), N, jnp.int32)]).reshape(gtot, GROUP)

    p1 = _make_pass1(n_pad, gtot, gp_w)
    p2 = _make_pass2(n_pad, gtot, gp_w, H)

    def layer(h, el, er, gm):
        el_p = jnp.pad(el[:, 0], (0, n_pad - N))
        er_p = jnp.pad(er[:, 0], (0, n_pad - N))
        ex, dpart = p1(src2d, dst2d, el_p, er_p, gm)
        dn = _dsum(dpart).reshape(n_pad)
        op = p2(src2d, dst2d, ex, dn, h)
        return op[:, :N]

    h, el, er, gm = _dense(feat, W1, al1, ar1)
    op = layer(h, el, er, gm)
    h, el, er, gm = _combine_dense(op[0], op[1], b1, W2, al2, ar2)
    op = layer(h, el, er, gm)
    h, el, er, gm = _combine_dense(op[0], op[1], b2, W3, al3, ar3)
    op = layer(h, el, er, gm)
    out = _combine(op[0], op[1], b3)
    return out[:, None, :]
